# baseline (speedup 1.0000x reference)
import jax
import jax.numpy as jnp
from jax.experimental import pallas as pl
from jax.experimental.pallas import tpu as pltpu
from jax.scipy.linalg import block_diag

BN = 8
GRP = 4


def _fluid_solid_kernel(mask_ref, cdTo_ref, cT_ref,
                        W0b_ref, Wc_ref, b0_ref,
                        W1b_ref, W2b_ref, W3g_ref,
                        fb3_ref, sb3_ref, out_ref, w0_scr):
    cdTo = cdTo_ref[...]
    m = cdTo.shape[1]
    ptype = cdTo[7:8, :]
    tf_row = 1.0 - ptype
    ts_row = ptype

    bfs = jnp.dot(Wc_ref[...], cT_ref[0]) + b0_ref[...]

    mask_blk = mask_ref[...]
    wf = mask_blk * tf_row
    ws = mask_blk * ts_row

    ones_row = jnp.ones((1, m), jnp.float32)
    outs = []
    for g in range(BN // GRP):
        w0_scr[g] = W0b_ref[...]
        for i in range(GRP):
            w0_scr[g, 36 * i:36 * (i + 1), 8 * i:8 * i + 1] = (
                bfs[:, g * GRP + i:g * GRP + i + 1])
        x0 = jnp.tanh(jnp.dot(w0_scr[g], cdTo))
        x0 = jnp.concatenate([x0, ones_row], axis=0)
        x1 = jnp.tanh(jnp.dot(W1b_ref[...], x0))
        x1 = jnp.concatenate([x1, ones_row], axis=0)
        x2 = jnp.tanh(jnp.dot(W2b_ref[...], x1))

        wfg = wf[g * GRP:(g + 1) * GRP]
        wsg = ws[g * GRP:(g + 1) * GRP]
        wsel = jnp.concatenate(
            [jnp.broadcast_to(wfg[:, None, :], (GRP, 6, m)),
             jnp.broadcast_to(wsg[:, None, :], (GRP, 6, m))],
            axis=1).reshape(12 * GRP, m)
        s = jnp.sum(x2 * wsel, axis=1, keepdims=True)
        outs.append(jnp.dot(W3g_ref[...], s).reshape(GRP, 3))
    out = jnp.concatenate(outs, axis=0)

    cf = jnp.sum(wf, axis=1, keepdims=True)
    cs = jnp.sum(ws, axis=1, keepdims=True)
    out_ref[0] = out + cf * fb3_ref[...] + cs * sb3_ref[...]


def kernel(mask, center_particle, current_data,
           fW0, fb0, fW1, fb1, fW2, fb2, fW3, fb3,
           sW0, sb0, sW1, sb1, sW2, sb2, sW3, sb3):
    n, m = mask.shape
    maskf = mask.astype(jnp.float32)
    cdT = current_data.T
    grid = n // BN
    cTb = center_particle.T.reshape(6, grid, BN).transpose(1, 0, 2)

    cdTo = jnp.tile(jnp.concatenate([jnp.ones((1, m), jnp.float32), cdT],
                                    axis=0), (GRP, 1))

    Wa = jnp.concatenate(
        [fW0[0:6].T,
         jnp.concatenate([sW0[0:3].T, jnp.zeros((18, 3), jnp.float32)],
                         axis=1)], axis=0)
    blk0 = jnp.zeros((36, 8), jnp.float32).at[:, 1:7].set(Wa)
    W0b = block_diag(*([blk0] * GRP))
    Wc = jnp.concatenate(
        [jnp.concatenate([-fW0[0:3].T, fW0[6:9].T], axis=1),
         jnp.concatenate([-sW0[0:3].T, sW0[3:6].T], axis=1)], axis=0)
    b0 = jnp.concatenate([fb0, sb0]).reshape(36, 1)

    W1fs = block_diag(fW1.T, sW1.T)
    W1g = block_diag(*([W1fs] * GRP))
    b1 = jnp.tile(jnp.concatenate([fb1, sb1]), GRP).reshape(18 * GRP, 1)
    W1b = jnp.concatenate([W1g, b1], axis=1)
    W2fs = block_diag(fW2.T, sW2.T)
    W2g = block_diag(*([W2fs] * GRP))
    b2 = jnp.tile(jnp.concatenate([fb2, sb2]), GRP).reshape(12 * GRP, 1)
    W2b = jnp.concatenate([W2g, b2], axis=1)
    W3fs = jnp.concatenate([fW3.T, sW3.T], axis=1)
    W3g = block_diag(*([W3fs] * GRP))

    full = lambda shape: pl.BlockSpec(shape, lambda i: tuple(0 for _ in shape))
    out = pl.pallas_call(
        _fluid_solid_kernel,
        grid=(grid,),
        in_specs=[
            pl.BlockSpec((BN, m), lambda i: (i, 0)),
            full((8 * GRP, m)),
            pl.BlockSpec((1, 6, BN), lambda i: (i, 0, 0)),
            full((36 * GRP, 8 * GRP)), full((36, 6)), full((36, 1)),
            full((18 * GRP, 36 * GRP + 1)),
            full((12 * GRP, 18 * GRP + 1)),
            full((12, 12 * GRP)),
            full((1, 3)), full((1, 3)),
        ],
        out_specs=pl.BlockSpec((1, BN, 3), lambda i: (i, 0, 0)),
        out_shape=jax.ShapeDtypeStruct((grid, BN, 3), jnp.float32),
        scratch_shapes=[pltpu.VMEM((BN // GRP, 36 * GRP, 8 * GRP),
                                   jnp.float32)],
    )(maskf, cdTo, cTb, W0b, Wc, b0, W1b, W2b, W3g,
      fb3.reshape(1, 3), sb3.reshape(1, 3))
    return out.reshape(n, 3)


# speedup vs baseline: 1.4387x; 1.4387x over previous
import jax
import jax.numpy as jnp
from jax.experimental import pallas as pl
from jax.scipy.linalg import block_diag

BN = 16
GRP = 4
R0 = 40
R2 = 16


def _fluid_solid_kernel(mask_ref, cdT_ref, cT_ref,
                        Wa_ref, Wc_ref, b0_ref,
                        W1b_ref, W2b_ref, W3g_ref,
                        fb3_ref, sb3_ref, out_ref):
    cdT = cdT_ref[...]
    m = cdT.shape[1]
    ptype = cdT[6:7, :]
    tf_row = 1.0 - ptype
    ts_row = ptype

    afs = jnp.dot(Wa_ref[...], cdT[0:6, :])
    bfs = jnp.dot(Wc_ref[...], cT_ref[0]) + b0_ref[...]

    mask_blk = mask_ref[...]

    zsel = jnp.zeros((1, R2 - 12, m), jnp.float32)
    tsel = jnp.concatenate(
        [jnp.broadcast_to(tf_row[:, None, :], (1, 6, m)),
         jnp.broadcast_to(ts_row[:, None, :], (1, 6, m)), zsel],
        axis=1).reshape(R2, m)
    tsel = jnp.concatenate([tsel] * GRP, axis=0)

    ones_row = jnp.ones((1, m), jnp.float32)
    outs = []
    for g in range(BN // GRP):
        x0 = jnp.concatenate(
            [afs + bfs[:, g * GRP + i:g * GRP + i + 1] for i in range(GRP)]
            + [ones_row], axis=0)
        x0 = jnp.tanh(x0)
        x1 = jnp.tanh(jnp.dot(W1b_ref[...], x0))
        x1 = jnp.concatenate([x1, ones_row], axis=0)
        x2 = jnp.tanh(jnp.dot(W2b_ref[...], x1))

        mrep = jnp.broadcast_to(
            mask_blk[g * GRP:(g + 1) * GRP, None, :],
            (GRP, R2, m)).reshape(R2 * GRP, m)
        s = jnp.sum(x2 * tsel * mrep, axis=1, keepdims=True)
        outs.append(jnp.dot(W3g_ref[...], s).reshape(GRP, 3))
    out = jnp.concatenate(outs, axis=0)

    cf = jnp.sum(mask_blk * tf_row, axis=1, keepdims=True)
    cs = jnp.sum(mask_blk * ts_row, axis=1, keepdims=True)
    out_ref[0] = out + cf * fb3_ref[...] + cs * sb3_ref[...]


def kernel(mask, center_particle, current_data,
           fW0, fb0, fW1, fb1, fW2, fb2, fW3, fb3,
           sW0, sb0, sW1, sb1, sW2, sb2, sW3, sb3):
    n, m = mask.shape
    maskf = mask.astype(jnp.float32)
    cdT = current_data.T
    grid = n // BN
    cTb = center_particle.T.reshape(6, grid, BN).transpose(1, 0, 2)

    z = lambda r, c: jnp.zeros((r, c), jnp.float32)
    Wa = jnp.concatenate(
        [fW0[0:6].T,
         jnp.concatenate([sW0[0:3].T, z(18, 3)], axis=1),
         z(R0 - 36, 6)], axis=0)
    Wc = jnp.concatenate(
        [jnp.concatenate([-fW0[0:3].T, fW0[6:9].T], axis=1),
         jnp.concatenate([-sW0[0:3].T, sW0[3:6].T], axis=1),
         z(R0 - 36, 6)], axis=0)
    b0 = jnp.concatenate([fb0, sb0, jnp.zeros(R0 - 36)]).reshape(R0, 1)

    W1fs = jnp.concatenate([block_diag(fW1.T, sW1.T), z(18, R0 - 36)],
                           axis=1)
    W1g = block_diag(*([W1fs] * GRP))
    b1 = jnp.tile(jnp.concatenate([fb1, sb1]), GRP).reshape(18 * GRP, 1)
    W1b = jnp.concatenate([W1g, b1 / jnp.tanh(1.0)], axis=1)
    W2fs = jnp.concatenate([block_diag(fW2.T, sW2.T), z(R2 - 12, 18)],
                           axis=0)
    W2g = block_diag(*([W2fs] * GRP))
    b2 = jnp.tile(jnp.concatenate([fb2, sb2, jnp.zeros(R2 - 12)]),
                  GRP).reshape(R2 * GRP, 1)
    W2b = jnp.concatenate([W2g, b2], axis=1)
    W3fs = jnp.concatenate([fW3.T, sW3.T, z(3, R2 - 12)], axis=1)
    W3g = block_diag(*([W3fs] * GRP))

    full = lambda shape: pl.BlockSpec(shape, lambda i: tuple(0 for _ in shape))
    out = pl.pallas_call(
        _fluid_solid_kernel,
        grid=(grid,),
        in_specs=[
            pl.BlockSpec((BN, m), lambda i: (i, 0)),
            full((7, m)),
            pl.BlockSpec((1, 6, BN), lambda i: (i, 0, 0)),
            full((R0, 6)), full((R0, 6)), full((R0, 1)),
            full((18 * GRP, R0 * GRP + 1)),
            full((R2 * GRP, 18 * GRP + 1)),
            full((3 * GRP, R2 * GRP)),
            full((1, 3)), full((1, 3)),
        ],
        out_specs=pl.BlockSpec((1, BN, 3), lambda i: (i, 0, 0)),
        out_shape=jax.ShapeDtypeStruct((grid, BN, 3), jnp.float32),
    )(maskf, cdT, cTb, Wa, Wc, b0, W1b, W2b, W3g,
      fb3.reshape(1, 3), sb3.reshape(1, 3))
    return out.reshape(n, 3)


# speedup vs baseline: 1.6206x; 1.1264x over previous
import jax
import jax.numpy as jnp
from jax.experimental import pallas as pl
from jax.scipy.linalg import block_diag

BN = 32
GRP = 4
R0 = 40
R2 = 16


def _fluid_solid_kernel(mask_ref, cdT_ref, cT_ref,
                        Wa_ref, Wc_ref, b0_ref,
                        W1b_ref, W2b_ref, W3g_ref,
                        fb3_ref, sb3_ref, out_ref):
    cdT = cdT_ref[...]
    m = cdT.shape[1]
    ptype = cdT[6:7, :]
    tf_row = 1.0 - ptype
    ts_row = ptype

    afs = jnp.dot(Wa_ref[...], cdT[0:6, :])
    bfs = jnp.dot(Wc_ref[...], cT_ref[0]) + b0_ref[...]

    mask_blk = mask_ref[...].astype(jnp.float32)

    wf = mask_blk * tf_row
    ws = mask_blk * ts_row

    zsel = jnp.zeros((GRP, R2 - 12, m), jnp.float32)
    ones_row = jnp.ones((1, m), jnp.float32)
    outs = []
    for g in range(BN // GRP):
        x0 = jnp.concatenate(
            [afs + bfs[:, g * GRP + i:g * GRP + i + 1] for i in range(GRP)]
            + [ones_row], axis=0)
        x0 = jnp.tanh(x0)
        x1 = jnp.tanh(jnp.dot(W1b_ref[...], x0))
        x1 = jnp.concatenate([x1, ones_row], axis=0)
        x2 = jnp.tanh(jnp.dot(W2b_ref[...], x1))

        wsel = jnp.concatenate(
            [jnp.broadcast_to(wf[g * GRP:(g + 1) * GRP, None, :], (GRP, 6, m)),
             jnp.broadcast_to(ws[g * GRP:(g + 1) * GRP, None, :], (GRP, 6, m)),
             zsel], axis=1).reshape(R2 * GRP, m)
        s = jnp.sum(x2 * wsel, axis=1, keepdims=True)
        outs.append(jnp.dot(W3g_ref[...], s).reshape(GRP, 3))
    out = jnp.concatenate(outs, axis=0)

    cf = jnp.sum(wf, axis=1, keepdims=True)
    cs = jnp.sum(ws, axis=1, keepdims=True)
    out_ref[0] = out + cf * fb3_ref[...] + cs * sb3_ref[...]


def kernel(mask, center_particle, current_data,
           fW0, fb0, fW1, fb1, fW2, fb2, fW3, fb3,
           sW0, sb0, sW1, sb1, sW2, sb2, sW3, sb3):
    n, m = mask.shape
    maskf = mask.astype(jnp.int8)
    cdT = current_data.T
    grid = n // BN
    cTb = center_particle.T.reshape(6, grid, BN).transpose(1, 0, 2)

    z = lambda r, c: jnp.zeros((r, c), jnp.float32)
    Wa = jnp.concatenate(
        [fW0[0:6].T,
         jnp.concatenate([sW0[0:3].T, z(18, 3)], axis=1),
         z(R0 - 36, 6)], axis=0)
    Wc = jnp.concatenate(
        [jnp.concatenate([-fW0[0:3].T, fW0[6:9].T], axis=1),
         jnp.concatenate([-sW0[0:3].T, sW0[3:6].T], axis=1),
         z(R0 - 36, 6)], axis=0)
    b0 = jnp.concatenate([fb0, sb0, jnp.zeros(R0 - 36)]).reshape(R0, 1)

    W1fs = jnp.concatenate([block_diag(fW1.T, sW1.T), z(18, R0 - 36)],
                           axis=1)
    W1g = block_diag(*([W1fs] * GRP))
    b1 = jnp.tile(jnp.concatenate([fb1, sb1]), GRP).reshape(18 * GRP, 1)
    W1b = jnp.concatenate([W1g, b1 / jnp.tanh(1.0)], axis=1)
    W2fs = jnp.concatenate([block_diag(fW2.T, sW2.T), z(R2 - 12, 18)],
                           axis=0)
    W2g = block_diag(*([W2fs] * GRP))
    b2 = jnp.tile(jnp.concatenate([fb2, sb2, jnp.zeros(R2 - 12)]),
                  GRP).reshape(R2 * GRP, 1)
    W2b = jnp.concatenate([W2g, b2], axis=1)
    W3fs = jnp.concatenate([fW3.T, sW3.T, z(3, R2 - 12)], axis=1)
    W3g = block_diag(*([W3fs] * GRP))

    full = lambda shape: pl.BlockSpec(shape, lambda i: tuple(0 for _ in shape))
    out = pl.pallas_call(
        _fluid_solid_kernel,
        grid=(grid,),
        in_specs=[
            pl.BlockSpec((BN, m), lambda i: (i, 0)),
            full((7, m)),
            pl.BlockSpec((1, 6, BN), lambda i: (i, 0, 0)),
            full((R0, 6)), full((R0, 6)), full((R0, 1)),
            full((18 * GRP, R0 * GRP + 1)),
            full((R2 * GRP, 18 * GRP + 1)),
            full((3 * GRP, R2 * GRP)),
            full((1, 3)), full((1, 3)),
        ],
        out_specs=pl.BlockSpec((1, BN, 3), lambda i: (i, 0, 0)),
        out_shape=jax.ShapeDtypeStruct((grid, BN, 3), jnp.float32),
    )(maskf, cdT, cTb, Wa, Wc, b0, W1b, W2b, W3g,
      fb3.reshape(1, 3), sb3.reshape(1, 3))
    return out.reshape(n, 3)
